# baseline (speedup 1.0000x reference)
import functools

import jax
import jax.numpy as jnp
from jax import lax
from jax.experimental import pallas as pl
from jax.experimental.pallas import tpu as pltpu
from jax.experimental.pallas import tpu_sc as plsc

NC, NS, L = 2, 16, 16
NW = NC * NS
B, N, S = 64, 128, 2
HW = 128 * 128
BPW = B // NW
PW = BPW * N
PWI = PW * S
GW = 128


def _sc_gather(out_flat, ind_flat, mask_flat):
    mesh = plsc.VectorSubcoreMesh(
        core_axis_name="c", subcore_axis_name="s",
        num_cores=NC, num_subcores=NS)

    @functools.partial(
        pl.kernel,
        out_type=[jax.ShapeDtypeStruct((B * N * S,), jnp.float32),
                  jax.ShapeDtypeStruct((B * N * S,), jnp.float32)],
        mesh=mesh,
        scratch_types=[
            pltpu.VMEM((PW,), jnp.int32),
            pltpu.VMEM((PW,), jnp.float32),
            pltpu.VMEM((PW,), jnp.int32),
            pltpu.VMEM((PW,), jnp.int32),
            pltpu.VMEM((PW,), jnp.float32),
            pltpu.VMEM((PW,), jnp.float32),
            pltpu.VMEM((2 * S, GW), jnp.int32),
            pltpu.SemaphoreType.DMA,
            pltpu.SemaphoreType.DMA,
            pltpu.SemaphoreType.DMA,
        ],
    )
    def k(out_hbm, ind_hbm, mask_hbm, pm_hbm, me_hbm,
          ind_v, m_v, pix0_v, pix1_v, pm0_v, pm1_v, six_v, sem, msem, wsem):
        wid = lax.axis_index("s") * NC + lax.axis_index("c")
        b0 = wid * BPW
        p0 = wid * PW
        q0 = wid * PWI
        iota = lax.iota(jnp.int32, L)
        ci = pltpu.async_copy(ind_hbm.at[pl.ds(p0, PW)], ind_v, sem)
        cm = pltpu.async_copy(mask_hbm.at[pl.ds(p0, PW)], m_v, msem)
        for r in range(S):
            for p in range(S):
                for c in range(GW // L):
                    six_v[r + S * p, pl.ds(c * L, L)] = (
                        iota * 2 + (q0 + 2 * (r * GW + c * L) + p))
        ci.wait()
        cpb = PW // N
        for j in range(BPW):
            base = (b0 + j) * (S * HW)
            for i in range(N // L):
                sl = pl.ds(j * N + i * L, L)
                c = ind_v[sl]
                pix0_v[sl] = c + base
                pix1_v[sl] = c + (base + HW)
        gs = []
        for w in range(PW // GW):
            win = pl.ds(w * GW, GW)
            gs.append(pltpu.async_copy(
                out_hbm.at[pix0_v.at[win]], pm0_v.at[win], sem))
            gs.append(pltpu.async_copy(
                out_hbm.at[pix1_v.at[win]], pm1_v.at[win], sem))
        cm.wait()
        ws = []
        for w in range(PW // GW):
            win = pl.ds(w * GW, GW)
            ws.append(pltpu.async_copy(
                m_v.at[win], me_hbm.at[six_v.at[w]], msem))
            ws.append(pltpu.async_copy(
                m_v.at[win], me_hbm.at[six_v.at[w + S]], msem))
        for g in gs:
            g.wait()
        for i in range(PW // L):
            sl = pl.ds(i * L, L)
            m = m_v[sl]
            pm0_v[sl] = pm0_v[sl] * m
            pm1_v[sl] = pm1_v[sl] * m
        for w in range(PW // GW):
            win = pl.ds(w * GW, GW)
            ws.append(pltpu.async_copy(
                pm0_v.at[win], pm_hbm.at[six_v.at[w]], wsem))
            ws.append(pltpu.async_copy(
                pm1_v.at[win], pm_hbm.at[six_v.at[w + S]], wsem))
        for g in ws:
            g.wait()

    return k(out_flat, ind_flat, mask_flat)


def _tc_loss(pm2, me2, tgt2, mask):
    def k(pm_ref, me_ref, t_ref, m_ref, o_ref):
        num = jnp.sum(jnp.abs(pm_ref[...] - t_ref[...] * me_ref[...]),
                      keepdims=True)
        den = jnp.sum(m_ref[...], keepdims=True) + 0.0001
        o_ref[...] = num / den

    return pl.pallas_call(
        k, out_shape=jax.ShapeDtypeStruct((1, 1), jnp.float32),
    )(pm2, me2, tgt2, mask)


def kernel(out, target, ind, mask):
    pm, me = _sc_gather(out.reshape(-1), ind.reshape(-1), mask.reshape(-1))
    r = _tc_loss(pm.reshape(N, N), me.reshape(N, N), target.reshape(N, N),
                 mask)
    return r.reshape(())


# speedup vs baseline: 5.8674x; 5.8674x over previous
import functools

import jax
import jax.numpy as jnp
from jax import lax
from jax.experimental import pallas as pl
from jax.experimental.pallas import tpu as pltpu
from jax.experimental.pallas import tpu_sc as plsc

NC, NS, L = 2, 16, 16
NW = NC * NS
B, N, S = 64, 128, 2
HW = 128 * 128
BPW = B // NW
PW = BPW * N
GW = 128


def _sc_gather(out_flat, ind_flat):
    mesh = plsc.VectorSubcoreMesh(
        core_axis_name="c", subcore_axis_name="s",
        num_cores=NC, num_subcores=NS)

    @functools.partial(
        pl.kernel,
        out_type=jax.ShapeDtypeStruct((S, B * N), jnp.float32),
        mesh=mesh,
        scratch_types=[
            pltpu.VMEM((PW,), jnp.int32),
            pltpu.VMEM((PW,), jnp.int32),
            pltpu.VMEM((PW,), jnp.int32),
            pltpu.VMEM((PW,), jnp.float32),
            pltpu.VMEM((PW,), jnp.float32),
            pltpu.SemaphoreType.DMA,
            pltpu.SemaphoreType.DMA,
        ],
    )
    def k(out_hbm, ind_hbm, pred_hbm,
          ind_v, pix0_v, pix1_v, p0_v, p1_v, sem, wsem):
        wid = lax.axis_index("s") * NC + lax.axis_index("c")
        b0 = wid * BPW
        p0 = wid * PW
        pltpu.sync_copy(ind_hbm.at[pl.ds(p0, PW)], ind_v)
        for j in range(BPW):
            base = (b0 + j) * (S * HW)
            for i in range(N // L):
                sl = pl.ds(j * N + i * L, L)
                c = ind_v[sl]
                pix0_v[sl] = c + base
                pix1_v[sl] = c + (base + HW)
        gs = []
        for w in range(PW // GW):
            win = pl.ds(w * GW, GW)
            gs.append(pltpu.async_copy(
                out_hbm.at[pix0_v.at[win]], p0_v.at[win], sem))
            gs.append(pltpu.async_copy(
                out_hbm.at[pix1_v.at[win]], p1_v.at[win], sem))
        for g in gs:
            g.wait()
        w0 = pltpu.async_copy(p0_v, pred_hbm.at[0, pl.ds(p0, PW)], wsem)
        w1 = pltpu.async_copy(p1_v, pred_hbm.at[1, pl.ds(p0, PW)], wsem)
        w0.wait()
        w1.wait()

    return k(out_flat, ind_flat)


def _tc_loss(pred, tgt_planes, mask):
    def k(p_ref, t_ref, m_ref, o_ref):
        p = p_ref[...]
        t = t_ref[...]
        m = m_ref[...]
        num = jnp.sum(jnp.abs(p * m - t * m), keepdims=True)
        den = jnp.sum(m, keepdims=True) + 0.0001
        o_ref[...] = num[0] / den

    return pl.pallas_call(
        k, out_shape=jax.ShapeDtypeStruct((1, 1), jnp.float32),
    )(pred, tgt_planes, mask)


def kernel(out, target, ind, mask):
    pred = _sc_gather(out.reshape(-1), ind.reshape(-1))
    tgt_planes = jnp.moveaxis(target, 2, 0)
    r = _tc_loss(pred.reshape(S, B, N), tgt_planes, mask)
    return r.reshape(())


# speedup vs baseline: 6.2737x; 1.0693x over previous
import functools

import jax
import jax.numpy as jnp
from jax import lax
from jax.experimental import pallas as pl
from jax.experimental.pallas import tpu as pltpu
from jax.experimental.pallas import tpu_sc as plsc

NC, NS, L = 2, 16, 16
NW = NC * NS
B, N, S = 64, 128, 2
HW = 128 * 128
BPW = B // NW
PW = BPW * N
GW = 128


def _sc_gather(out_flat, ind_flat):
    mesh = plsc.VectorSubcoreMesh(
        core_axis_name="c", subcore_axis_name="s",
        num_cores=NC, num_subcores=NS)

    @functools.partial(
        pl.kernel,
        out_type=jax.ShapeDtypeStruct((S * B * N,), jnp.float32),
        mesh=mesh,
        scratch_types=[
            pltpu.VMEM((PW,), jnp.int32),
            pltpu.VMEM((PW,), jnp.int32),
            pltpu.VMEM((PW,), jnp.int32),
            pltpu.VMEM((PW,), jnp.float32),
            pltpu.VMEM((PW,), jnp.float32),
            pltpu.SemaphoreType.DMA,
            pltpu.SemaphoreType.DMA,
        ],
    )
    def k(out_hbm, ind_hbm, pred_hbm,
          ind_v, pix0_v, pix1_v, p0_v, p1_v, sem, wsem):
        wid = lax.axis_index("s") * NC + lax.axis_index("c")
        b0 = wid * BPW
        p0 = wid * PW
        pltpu.sync_copy(ind_hbm.at[pl.ds(p0, PW)], ind_v)
        for j in range(BPW):
            base = (b0 + j) * (S * HW)
            for i in range(N // L):
                sl = pl.ds(j * N + i * L, L)
                c = ind_v[sl]
                pix0_v[sl] = c + base
                pix1_v[sl] = c + (base + HW)
        gs = []
        for w in range(PW // GW):
            win = pl.ds(w * GW, GW)
            gs.append(pltpu.async_copy(
                out_hbm.at[pix0_v.at[win]], p0_v.at[win], sem))
            gs.append(pltpu.async_copy(
                out_hbm.at[pix1_v.at[win]], p1_v.at[win], sem))
        for g in gs:
            g.wait()
        w0 = pltpu.async_copy(p0_v, pred_hbm.at[pl.ds(p0, PW)], wsem)
        w1 = pltpu.async_copy(p1_v, pred_hbm.at[pl.ds(B * N + p0, PW)], wsem)
        w0.wait()
        w1.wait()

    return k(out_flat, ind_flat)


def _tc_loss(pred, tgt_planes, mask):
    def k(p_ref, t_ref, m_ref, o_ref):
        p = p_ref[...]
        t = t_ref[...]
        m = m_ref[...]
        num = jnp.sum(jnp.abs(p * m - t * m), keepdims=True)
        den = jnp.sum(m, keepdims=True) + 0.0001
        o_ref[...] = num[0] / den

    return pl.pallas_call(
        k, out_shape=jax.ShapeDtypeStruct((1, 1), jnp.float32),
    )(pred, tgt_planes, mask)


def kernel(out, target, ind, mask):
    pred = _sc_gather(out.reshape(-1), ind.reshape(-1))
    tgt_planes = jnp.moveaxis(target, 2, 0)
    r = _tc_loss(pred.reshape(S, B, N), tgt_planes, mask)
    return r.reshape(())


# speedup vs baseline: 6.2965x; 1.0036x over previous
import functools

import jax
import jax.numpy as jnp
from jax import lax
from jax.experimental import pallas as pl
from jax.experimental.pallas import tpu as pltpu
from jax.experimental.pallas import tpu_sc as plsc

NC, NS, L = 2, 16, 16
NW = NC * NS
B, N, S = 64, 128, 2
HW = 128 * 128
BPW = B // NW
PW = BPW * N
GW = 128


def _sc_gather(out_flat, ind_flat):
    mesh = plsc.VectorSubcoreMesh(
        core_axis_name="c", subcore_axis_name="s",
        num_cores=NC, num_subcores=NS)

    @functools.partial(
        pl.kernel,
        out_type=jax.ShapeDtypeStruct((S * B * N // GW, GW), jnp.float32),
        mesh=mesh,
        scratch_types=[
            pltpu.VMEM((PW,), jnp.int32),
            pltpu.VMEM((2 * S, GW), jnp.int32),
            pltpu.VMEM((2 * S, GW), jnp.float32),
            pltpu.SemaphoreType.DMA,
            pltpu.SemaphoreType.DMA,
        ],
    )
    def k(out_hbm, ind_hbm, pred_hbm, ind_v, pix_v, p_v, sem, wsem):
        wid = lax.axis_index("s") * NC + lax.axis_index("c")
        b0 = wid * BPW
        p0 = wid * PW
        pltpu.sync_copy(ind_hbm.at[pl.ds(p0, PW)], ind_v)
        for j in range(BPW):
            base = (b0 + j) * (S * HW)
            for i in range(N // L):
                c = ind_v[pl.ds(j * N + i * L, L)]
                pix_v[j, pl.ds(i * L, L)] = c + base
                pix_v[S + j, pl.ds(i * L, L)] = c + (base + HW)
        gs = [pltpu.async_copy(out_hbm.at[pix_v.at[r]], p_v.at[r], sem)
              for r in range(2 * S)]
        for g in gs:
            g.wait()
        w0 = pltpu.async_copy(
            p_v.at[pl.ds(0, S)], pred_hbm.at[pl.ds(S * wid, S)], wsem)
        w1 = pltpu.async_copy(
            p_v.at[pl.ds(S, S)],
            pred_hbm.at[pl.ds(B * N // GW + S * wid, S)], wsem)
        w0.wait()
        w1.wait()

    return k(out_flat, ind_flat)


def _tc_loss(pred, tgt_planes, mask):
    def k(p_ref, t_ref, m_ref, o_ref):
        p = p_ref[...]
        t = t_ref[...]
        m = m_ref[...]
        num = jnp.sum(jnp.abs(p * m - t * m), keepdims=True)
        den = jnp.sum(m, keepdims=True) + 0.0001
        o_ref[...] = num[0] / den

    return pl.pallas_call(
        k, out_shape=jax.ShapeDtypeStruct((1, 1), jnp.float32),
    )(pred, tgt_planes, mask)


def kernel(out, target, ind, mask):
    pred = _sc_gather(out.reshape(-1), ind.reshape(-1))
    tgt_planes = jnp.moveaxis(target, 2, 0)
    r = _tc_loss(pred.reshape(S, B, N), tgt_planes, mask)
    return r.reshape(())
